# Initial kernel scaffold; baseline (speedup 1.0000x reference)
import jax
import jax.numpy as jnp
from jax.experimental import pallas as pl


def kernel(x, edge_index, edge_weight, attention, conv_z_W, conv_z_b, conv_r_W, conv_r_b, conv_h_W, conv_h_b, lin_z_W, lin_z_b, lin_r_W, lin_r_b, lin_h_W, lin_h_b, dec_W1, dec_b1, dec_W2, dec_b2, dec_W3, dec_b3):
    raise NotImplementedError("write your pallas kernel here")


# speedup vs baseline: 81.7023x; 81.7023x over previous
import functools
import jax
import jax.numpy as jnp
from jax import lax
from jax.experimental import pallas as pl
from jax.experimental.pallas import tpu as pltpu
from jax.experimental.pallas import tpu_sc as plsc

_N = 10000
_NPAD = 10240
_E = 320000
_F = 9
_HID = 32
_P = 10
_PRED = 50
_FP = 128
_K = 128
_NB = 79
_EPT = _NB * _K
_EPAD = 32 * _EPT
_RPT = _NPAD // 16
_BLK = 1024

@functools.lru_cache(maxsize=None)
def _sc_kernels():
    mesh = plsc.VectorSubcoreMesh(core_axis_name="c", subcore_axis_name="s")

    @functools.partial(
        pl.kernel,
        out_type=jax.ShapeDtypeStruct((2, _NPAD), jnp.float32),
        mesh=mesh,
        scratch_types=[
            pltpu.VMEM((_K,), jnp.int32),
            pltpu.VMEM((_K,), jnp.float32),
            pltpu.VMEM((_RPT,), jnp.float32),
            pltpu.VMEM_SHARED((_NPAD,), jnp.float32),
        ],
    )
    def _deg_kernel(col_hbm, w_hbm, deg_out, colbuf, wbuf, zbuf, deg_sh):
        c = lax.axis_index("c")
        s = lax.axis_index("s")
        wid = c * 16 + s
        for i in range(_RPT // 16):
            zbuf[pl.ds(i * 16, 16)] = jnp.zeros((16,), jnp.float32)
        r0 = s * _RPT
        pltpu.sync_copy(zbuf, deg_sh.at[pl.ds(r0, _RPT)])
        plsc.subcore_barrier()

        def body(b, carry):
            base = wid * _EPT + b * _K
            pltpu.sync_copy(col_hbm.at[pl.ds(base, _K)], colbuf)
            pltpu.sync_copy(w_hbm.at[pl.ds(base, _K)], wbuf)
            pltpu.sync_copy(wbuf, deg_sh.at[colbuf], add=True)
            return carry

        lax.fori_loop(0, _NB, body, 0)
        plsc.subcore_barrier()
        pltpu.sync_copy(deg_sh.at[pl.ds(r0, _RPT)],
                        deg_out.at[c, pl.ds(r0, _RPT)])

    @functools.partial(
        pl.kernel,
        out_type=jax.ShapeDtypeStruct((2, _NPAD, _FP), jnp.float32),
        mesh=mesh,
        scratch_types=[
            pltpu.VMEM((_K,), jnp.int32),
            pltpu.VMEM((_K,), jnp.int32),
            pltpu.VMEM((_K,), jnp.float32),
            pltpu.VMEM((_K, _FP), jnp.float32),
            pltpu.VMEM((_K, _FP), jnp.float32),
            pltpu.VMEM_SHARED((_NPAD, _FP), jnp.float32),
            pltpu.SemaphoreType.DMA,
        ],
    )
    def _spmm_kernel(row_hbm, col_hbm, w_hbm, xs_hbm, y_out,
                     rowbuf, colbuf, wbuf, xbuf, zbuf, y_sh, sem):
        c = lax.axis_index("c")
        s = lax.axis_index("s")
        wid = c * 16 + s

        def zrow(i, carry):
            for j in range(_FP // 16):
                zbuf[i, pl.ds(j * 16, 16)] = jnp.zeros((16,), jnp.float32)
            return carry

        lax.fori_loop(0, _K, zrow, 0)
        r0 = s * _RPT

        def zcopy(i, carry):
            pltpu.sync_copy(zbuf, y_sh.at[pl.ds(r0 + i * _K, _K), :])
            return carry

        lax.fori_loop(0, _RPT // _K, zcopy, 0)
        plsc.subcore_barrier()

        def body(b, carry):
            base = wid * _EPT + b * _K
            pltpu.sync_copy(row_hbm.at[pl.ds(base, _K)], rowbuf)
            pltpu.sync_copy(col_hbm.at[pl.ds(base, _K)], colbuf)
            pltpu.sync_copy(w_hbm.at[pl.ds(base, _K)], wbuf)
            pltpu.async_copy(xs_hbm.at[rowbuf], xbuf, sem).wait()

            def scale(g, c2):
                wv = wbuf[pl.ds(g * 16, 16)]
                for l in range(16):
                    w = wv[l]
                    e = g * 16 + l
                    for j in range(_FP // 16):
                        xbuf[e, pl.ds(j * 16, 16)] = \
                            xbuf[e, pl.ds(j * 16, 16)] * w
                return c2

            lax.fori_loop(0, _K // 16, scale, 0)
            pltpu.sync_copy(xbuf, y_sh.at[colbuf], add=True)
            return carry

        lax.fori_loop(0, _NB, body, 0)
        plsc.subcore_barrier()
        pltpu.sync_copy(y_sh.at[pl.ds(r0, _RPT), :],
                        y_out.at[c, pl.ds(r0, _RPT), :])

    return _deg_kernel, _spmm_kernel


def _scale_body(degt_ref, x_ref, xs_ref):
    d = degt_ref[:, 0:1] + degt_ref[:, 1:2] + 1.0
    dinv = lax.rsqrt(d)
    xs_ref[...] = x_ref[...] * dinv


def _dense_body(degt_ref, x_ref, ya_ref, yb_ref, bz_ref, bh_ref, bzb_ref,
                bhb_ref, pm_ref, w1_ref, b1_ref, w2_ref, b2_ref, w3_ref,
                b3_ref, out_ref):
    d = degt_ref[:, 0:1] + degt_ref[:, 1:2] + 1.0
    dinv = lax.rsqrt(d)
    y = dinv * (ya_ref[0] + yb_ref[0]) + (dinv * dinv) * x_ref[...]
    gz = jnp.dot(y, bz_ref[...], preferred_element_type=jnp.float32,
                   precision=lax.Precision.HIGHEST) + bzb_ref[...]
    gh = jnp.dot(y, bh_ref[...], preferred_element_type=jnp.float32,
                   precision=lax.Precision.HIGHEST) + bhb_ref[...]
    u = (1.0 - jax.nn.sigmoid(gz)) * jnp.tanh(gh)
    h = jnp.dot(u, pm_ref[...], preferred_element_type=jnp.float32,
                   precision=lax.Precision.HIGHEST)
    h = jax.nn.relu(h)
    h = jax.nn.relu(jnp.dot(h, w1_ref[...], preferred_element_type=jnp.float32,
                   precision=lax.Precision.HIGHEST)
                    + b1_ref[...])
    h = jax.nn.relu(jnp.dot(h, w2_ref[...], preferred_element_type=jnp.float32,
                   precision=lax.Precision.HIGHEST)
                    + b2_ref[...])
    out_ref[...] = jnp.dot(h, w3_ref[...], preferred_element_type=jnp.float32,
                   precision=lax.Precision.HIGHEST) \
        + b3_ref[...]


def kernel(x, edge_index, edge_weight, attention,
           conv_z_W, conv_z_b, conv_r_W, conv_r_b, conv_h_W, conv_h_b,
           lin_z_W, lin_z_b, lin_r_W, lin_r_b, lin_h_W, lin_h_b,
           dec_W1, dec_b1, dec_W2, dec_b2, dec_W3, dec_b3):
    n = x.shape[1]

    xflat = x[0].reshape(n, _F * _P)
    xpad = jnp.pad(xflat, ((0, _NPAD - n), (0, _FP - _F * _P)))
    npad_e = _EPAD - _E
    rowp = jnp.concatenate([edge_index[0], jnp.zeros((npad_e,), jnp.int32)])
    colp = jnp.concatenate([edge_index[1], jnp.zeros((npad_e,), jnp.int32)])
    wp = jnp.concatenate([edge_weight, jnp.zeros((npad_e,), jnp.float32)])

    lz = lin_z_W[:_HID]
    lh = lin_h_W[:_HID]
    az = conv_z_W @ lz
    ah = conv_h_W @ lh
    bz_bias = jnp.tile(conv_z_b @ lz + lin_z_b, _P)[None, :]
    bh_bias = jnp.tile(conv_h_b @ lh + lin_h_b, _P)[None, :]
    eye_p = jnp.eye(_P, dtype=jnp.float32)
    bz_w = (az[:, None, None, :] * eye_p[None, :, :, None]).reshape(
        _F * _P, _P * _HID)
    bh_w = (ah[:, None, None, :] * eye_p[None, :, :, None]).reshape(
        _F * _P, _P * _HID)
    bz_w = jnp.pad(bz_w, ((0, _FP - _F * _P), (0, 0)))
    bh_w = jnp.pad(bh_w, ((0, _FP - _F * _P), (0, 0)))
    probs = jax.nn.softmax(attention)
    pmat = jnp.repeat(probs, _HID)[:, None] * jnp.tile(
        jnp.eye(_HID, dtype=jnp.float32), (_P, 1))

    deg_k, spmm_k = _sc_kernels()
    deg2 = deg_k(colp, wp)
    degt = deg2.T

    nblk = _NPAD // _BLK
    xs = pl.pallas_call(
        _scale_body,
        grid=(nblk,),
        in_specs=[
            pl.BlockSpec((_BLK, 2), lambda i: (i, 0)),
            pl.BlockSpec((_BLK, _FP), lambda i: (i, 0)),
        ],
        out_specs=pl.BlockSpec((_BLK, _FP), lambda i: (i, 0)),
        out_shape=jax.ShapeDtypeStruct((_NPAD, _FP), jnp.float32),
    )(degt, xpad)

    y2 = spmm_k(rowp, colp, wp, xs)

    full = lambda shape: pl.BlockSpec(shape, lambda i: tuple(0 for _ in shape))
    out = pl.pallas_call(
        _dense_body,
        grid=(nblk,),
        in_specs=[
            pl.BlockSpec((_BLK, 2), lambda i: (i, 0)),
            pl.BlockSpec((_BLK, _FP), lambda i: (i, 0)),
            pl.BlockSpec((1, _BLK, _FP), lambda i: (0, i, 0)),
            pl.BlockSpec((1, _BLK, _FP), lambda i: (1, i, 0)),
            full((_FP, _P * _HID)),
            full((_FP, _P * _HID)),
            full((1, _P * _HID)),
            full((1, _P * _HID)),
            full((_P * _HID, _HID)),
            full((_HID, 2 * _HID)),
            full((1, 2 * _HID)),
            full((2 * _HID, _HID)),
            full((1, _HID)),
            full((_HID, 2 * _PRED)),
            full((1, 2 * _PRED)),
        ],
        out_specs=pl.BlockSpec((_BLK, 2 * _PRED), lambda i: (i, 0)),
        out_shape=jax.ShapeDtypeStruct((_NPAD, 2 * _PRED), jnp.float32),
    )(degt, xpad, y2, y2, bz_w, bh_w, bz_bias, bh_bias, pmat,
      dec_W1, dec_b1[None, :], dec_W2, dec_b2[None, :], dec_W3, dec_b3[None, :])

    return out[:n].reshape(n, _PRED, 2)
